# Initial kernel scaffold; baseline (speedup 1.0000x reference)
import jax
import jax.numpy as jnp
from jax.experimental import pallas as pl


def kernel(scores, labels, un_id):
    raise NotImplementedError("write your pallas kernel here")


# speedup vs baseline: 1.0016x; 1.0016x over previous
import jax
import jax.numpy as jnp
from jax.experimental import pallas as pl

_C = 81
_TOPK = 256
_N = 262144
_ROWS = 4096
_GRID = _N // _ROWS

_MINF = float("-inf")


def _stream_kernel(scores_ref, labels_ref, pos_ref, neg_ref, contrib_ref):
    s = scores_ref[...]
    lab = labels_ref[...]
    col = jax.lax.broadcasted_iota(jnp.int32, s.shape, 1)

    minf = jnp.float32(_MINF)
    m_all = jnp.max(jnp.where(col != _C - 1, s, minf), axis=1, keepdims=True)
    m_full = jnp.max(s, axis=1, keepdims=True)
    sumexp = jnp.sum(jnp.exp(s - m_full), axis=1, keepdims=True)
    lse = m_full + jnp.log(sumexp)

    s_lab = jnp.sum(jnp.where(col == lab, s, 0.0), axis=1, keepdims=True)
    s79 = s[:, _C - 2:_C - 1]
    s80 = s[:, _C - 1:_C]
    tgt = jnp.where(lab == _C - 1, s79, s80)

    gt = jnp.exp(s_lab - lse)
    soft = gt * (1.0 - gt)
    one_m = jnp.maximum(1.0 - gt, jnp.float32(1e-30))
    contrib = soft * (lse + jnp.log(one_m) - tgt)

    fg = lab != _C
    metric = -m_all
    pos_ref[...] = jnp.where(fg, metric, minf)
    neg_ref[...] = jnp.where(fg, minf, metric)
    contrib_ref[...] = contrib


def _f32_key(x):
    bits = jax.lax.bitcast_convert_type(x, jnp.int32)
    return bits ^ (jnp.right_shift(bits, 31) & jnp.int32(0x7FFFFFFF))


def _kth_threshold(keys, k):
    msb = jnp.int32(-2147483648)
    t = jnp.int32(0)
    for b in range(31, -1, -1):
        bit = msb if b == 31 else jnp.int32(1 << b)
        cand = t | bit
        cnt = jnp.sum((keys >= (cand ^ msb)).astype(jnp.int32))
        t = jnp.where(cnt >= k, cand, t)
    return t ^ msb


def _select_kernel(pos_ref, neg_ref, contrib_ref, out_ref):
    pos = pos_ref[...]
    neg = neg_ref[...]
    contrib = contrib_ref[...]

    minf = jnp.float32(_MINF)
    num_fg = jnp.sum((pos != minf).astype(jnp.int32))
    k = jnp.minimum(num_fg, jnp.int32(_TOPK))

    kpos = _f32_key(pos)
    kneg = _f32_key(neg)
    tpos = _kth_threshold(kpos, k)
    tneg = _kth_threshold(kneg, k)

    total = (jnp.sum(jnp.where(kpos >= tpos, contrib, 0.0))
             + jnp.sum(jnp.where(kneg >= tneg, contrib, 0.0)))
    loss = total / (k + k).astype(jnp.float32)
    out_ref[...] = jnp.full((1, 1), loss, dtype=jnp.float32)


def kernel(scores, labels, un_id):
    del un_id
    labels2 = labels.reshape(_N, 1).astype(jnp.int32)
    pos, neg, contrib = pl.pallas_call(
        _stream_kernel,
        grid=(_GRID,),
        in_specs=[
            pl.BlockSpec((_ROWS, _C + 1), lambda i: (i, 0)),
            pl.BlockSpec((_ROWS, 1), lambda i: (i, 0)),
        ],
        out_specs=[
            pl.BlockSpec((_ROWS, 1), lambda i: (i, 0)),
            pl.BlockSpec((_ROWS, 1), lambda i: (i, 0)),
            pl.BlockSpec((_ROWS, 1), lambda i: (i, 0)),
        ],
        out_shape=[
            jax.ShapeDtypeStruct((_N, 1), jnp.float32),
            jax.ShapeDtypeStruct((_N, 1), jnp.float32),
            jax.ShapeDtypeStruct((_N, 1), jnp.float32),
        ],
    )(scores, labels2)

    shape2 = (_N // 2048, 2048)
    out = pl.pallas_call(
        _select_kernel,
        out_shape=jax.ShapeDtypeStruct((1, 1), jnp.float32),
    )(pos.reshape(shape2), neg.reshape(shape2), contrib.reshape(shape2))
    return out[0, 0]


# speedup vs baseline: 1.1230x; 1.1212x over previous
import jax
import jax.numpy as jnp
from jax.experimental import pallas as pl

_C = 81
_TOPK = 256
_N = 262144
_ROWS = 4096
_SUB = _ROWS // 128
_GRID = _N // _ROWS

_MINF = float("-inf")


def _stream_kernel(scores_ref, labels_ref, pos_ref, neg_ref, contrib_ref):
    s = scores_ref[...].reshape(_SUB, 128, _C + 1)
    lab = labels_ref[0]
    col = jax.lax.broadcasted_iota(jnp.int32, s.shape, 2)

    minf = jnp.float32(_MINF)
    m_all = jnp.maximum(jnp.max(s[:, :, :_C - 1], axis=2), s[:, :, _C])
    m_full = jnp.maximum(m_all, s[:, :, _C - 1])
    sumexp = jnp.sum(jnp.exp(s - m_full[:, :, None]), axis=2)
    lse = m_full + jnp.log(sumexp)

    s_lab = jnp.sum(jnp.where(col == lab[:, :, None], s, 0.0), axis=2)
    tgt = jnp.where(lab == _C - 1, s[:, :, _C - 2], s[:, :, _C - 1])

    gt = jnp.exp(s_lab - lse)
    soft = gt * (1.0 - gt)
    one_m = jnp.maximum(1.0 - gt, jnp.float32(1e-30))
    contrib = soft * (lse + jnp.log(one_m) - tgt)

    fg = lab != _C
    metric = -m_all
    pos_ref[0] = jnp.where(fg, metric, minf)
    neg_ref[0] = jnp.where(fg, minf, metric)
    contrib_ref[0] = contrib


def _f32_key(x):
    bits = jax.lax.bitcast_convert_type(x, jnp.int32)
    return bits ^ (jnp.right_shift(bits, 31) & jnp.int32(0x7FFFFFFF))


def _dual_kth_threshold(kp, kn, k):
    msb = jnp.int32(-2147483648)
    tp = jnp.int32(0)
    tn = jnp.int32(0)
    for b in range(31, -1, -1):
        bit = msb if b == 31 else jnp.int32(1 << b)
        cp = tp | bit
        cn = tn | bit
        np_ = jnp.sum((kp >= (cp ^ msb)).astype(jnp.int32))
        nn_ = jnp.sum((kn >= (cn ^ msb)).astype(jnp.int32))
        tp = jnp.where(np_ >= k, cp, tp)
        tn = jnp.where(nn_ >= k, cn, tn)
    return tp ^ msb, tn ^ msb


def _select_kernel(pos_ref, neg_ref, contrib_ref, out_ref):
    pos = pos_ref[...]
    neg = neg_ref[...]
    contrib = contrib_ref[...]

    minf = jnp.float32(_MINF)
    num_fg = jnp.sum((pos != minf).astype(jnp.int32))
    k = jnp.minimum(num_fg, jnp.int32(_TOPK))

    kpos = _f32_key(pos)
    kneg = _f32_key(neg)
    tpos, tneg = _dual_kth_threshold(kpos, kneg, k)

    total = (jnp.sum(jnp.where(kpos >= tpos, contrib, 0.0))
             + jnp.sum(jnp.where(kneg >= tneg, contrib, 0.0)))
    loss = total / (k + k).astype(jnp.float32)
    out_ref[...] = jnp.full((1, 1), loss, dtype=jnp.float32)


def kernel(scores, labels, un_id):
    del un_id
    labels3 = labels.reshape(_GRID, _SUB, 128).astype(jnp.int32)
    pos, neg, contrib = pl.pallas_call(
        _stream_kernel,
        grid=(_GRID,),
        in_specs=[
            pl.BlockSpec((_ROWS, _C + 1), lambda i: (i, 0)),
            pl.BlockSpec((1, _SUB, 128), lambda i: (i, 0, 0)),
        ],
        out_specs=[
            pl.BlockSpec((1, _SUB, 128), lambda i: (i, 0, 0)),
            pl.BlockSpec((1, _SUB, 128), lambda i: (i, 0, 0)),
            pl.BlockSpec((1, _SUB, 128), lambda i: (i, 0, 0)),
        ],
        out_shape=[
            jax.ShapeDtypeStruct((_GRID, _SUB, 128), jnp.float32),
            jax.ShapeDtypeStruct((_GRID, _SUB, 128), jnp.float32),
            jax.ShapeDtypeStruct((_GRID, _SUB, 128), jnp.float32),
        ],
    )(scores, labels3)

    shape2 = (_N // 128, 128)
    out = pl.pallas_call(
        _select_kernel,
        out_shape=jax.ShapeDtypeStruct((1, 1), jnp.float32),
    )(pos.reshape(shape2), neg.reshape(shape2), contrib.reshape(shape2))
    return out[0, 0]
